# baseline (speedup 1.0000x reference)
import functools

import jax
import jax.numpy as jnp
import numpy as np
from jax import lax
from jax.experimental import pallas as pl
from jax.experimental.pallas import tpu as pltpu
from jax.experimental.pallas import tpu_sc as plsc

N = 2049
B = 8
TR = 512
PAD = 2064
NO = 2048

RTC = 1536
NT_FULL = RTC // TR
LAST_BLK = 2048 // TR

MATCH_THRESHOLD_F32 = np.float32(0.2)
BIG_I32 = np.int32(2**30)
NEG_INF = np.float32(-np.inf)

SC_CORES = 2
SC_SUBCORES = 16
SC_LANES = 16
NW = SC_CORES * SC_SUBCORES
WPR = NW // B
SROWS = NO - RTC
RPW = SROWS // WPR
RW = 16
NCHUNK = RPW // RW
NSEG = N // SC_LANES
TAIL = N - SC_LANES
CPW = NO // WPR


def _stage1a_body(x_ref, rowmax_ref, rowarg_ref, colmax_ref, colarg_ref):
    t = pl.program_id(2)
    x = x_ref[0, 0]

    col_ids = lax.broadcasted_iota(jnp.int32, (TR, N), 1)
    rmax = jnp.max(x, axis=1, keepdims=True)
    rarg = jnp.min(jnp.where(x == rmax, col_ids, BIG_I32), axis=1, keepdims=True)
    rowmax_ref[0, 0] = rmax.T
    rowarg_ref[0, 0] = rarg.T

    @pl.when(t == 0)
    def _():
        row_ids = lax.broadcasted_iota(jnp.int32, (TR, N), 0)
        cmax = jnp.max(x, axis=0, keepdims=True)
        carg = jnp.min(jnp.where(x == cmax, row_ids, BIG_I32), axis=0, keepdims=True)
        colmax_ref[0, 0] = cmax
        colarg_ref[0, 0] = carg

    @pl.when(jnp.logical_and(t > 0, t < NT_FULL))
    def _():
        row_ids = lax.broadcasted_iota(jnp.int32, (TR, N), 0) + t * TR
        cmax = jnp.max(x, axis=0, keepdims=True)
        carg = jnp.min(jnp.where(x == cmax, row_ids, BIG_I32), axis=0, keepdims=True)
        prev_max = colmax_ref[0, 0]
        prev_arg = colarg_ref[0, 0]
        upd = cmax > prev_max
        colmax_ref[0, 0] = jnp.where(upd, cmax, prev_max)
        colarg_ref[0, 0] = jnp.where(upd, carg, prev_arg)

    @pl.when(t == NT_FULL)
    def _():
        last = x[0:1, :]
        prev_max = colmax_ref[0, 0]
        prev_arg = colarg_ref[0, 0]
        upd = last > prev_max
        colmax_ref[0, 0] = jnp.where(upd, last, prev_max)
        colarg_ref[0, 0] = jnp.where(upd, jnp.full_like(prev_arg, N - 1), prev_arg)


def _row_blk(t):
    return jnp.where(t < NT_FULL, t, LAST_BLK)


_stage1a = pl.pallas_call(
    _stage1a_body,
    grid=(2, 4, NT_FULL + 1),
    in_specs=[pl.BlockSpec((1, 1, TR, N), lambda a, b, t: (a, b, _row_blk(t), 0))],
    out_specs=[
        pl.BlockSpec((1, 1, 1, TR), lambda a, b, t: (a, b, 0, _row_blk(t))),
        pl.BlockSpec((1, 1, 1, TR), lambda a, b, t: (a, b, 0, _row_blk(t))),
        pl.BlockSpec((1, 1, 1, N), lambda a, b, t: (a, b, 0, 0)),
        pl.BlockSpec((1, 1, 1, N), lambda a, b, t: (a, b, 0, 0)),
    ],
    out_shape=[
        jax.ShapeDtypeStruct((2, 4, 1, N), jnp.float32),
        jax.ShapeDtypeStruct((2, 4, 1, N), jnp.int32),
        jax.ShapeDtypeStruct((2, 4, 1, N), jnp.float32),
        jax.ShapeDtypeStruct((2, 4, 1, N), jnp.int32),
    ],
    compiler_params=pltpu.CompilerParams(
        dimension_semantics=("parallel", "parallel", "arbitrary"),
    ),
)


@functools.partial(
    pl.kernel,
    out_type=[
        jax.ShapeDtypeStruct((NW, PAD), jnp.float32),
        jax.ShapeDtypeStruct((NW, PAD), jnp.int32),
        jax.ShapeDtypeStruct((B, SROWS), jnp.float32),
        jax.ShapeDtypeStruct((B, SROWS), jnp.int32),
    ],
    mesh=plsc.VectorSubcoreMesh(core_axis_name="c", subcore_axis_name="s"),
    compiler_params=pltpu.CompilerParams(needs_layout_passes=False),
    scratch_types=[
        pltpu.VMEM((2, RW, N), jnp.float32),
        pltpu.VMEM((PAD,), jnp.float32),
        pltpu.VMEM((PAD,), jnp.int32),
        pltpu.VMEM((RPW,), jnp.float32),
        pltpu.VMEM((RPW,), jnp.int32),
        pltpu.SemaphoreType.DMA,
        pltpu.SemaphoreType.DMA,
    ],
)
def _stage1b(x_hbm, colpm_hbm, colpa_hbm, rowpm_hbm, rowpa_hbm,
             buf_v, cmax_v, carg_v, rpm_v, rpa_v, sem0, sem1):
    wid = lax.axis_index("s") * SC_CORES + lax.axis_index("c")
    p = wid // WPR
    q = wid % WPR
    a = p // 4
    b = p % 4
    r0 = RTC + q * RPW

    for c in range(PAD // SC_LANES):
        cmax_v[pl.ds(c * SC_LANES, SC_LANES)] = jnp.full((SC_LANES,), NEG_INF, jnp.float32)
        carg_v[pl.ds(c * SC_LANES, SC_LANES)] = jnp.zeros((SC_LANES,), jnp.int32)

    sems = (sem0, sem1)

    def start(k, j):
        pltpu.make_async_copy(
            x_hbm.at[a, b, pl.ds(r0 + k * RW, RW), :], buf_v.at[j], sems[j]).start()

    def wait(j):
        pltpu.make_async_copy(
            x_hbm.at[a, b, pl.ds(r0, RW), :], buf_v.at[j], sems[j]).wait()

    start(0, 0)
    start(np.int32(1), 1)

    lane = lax.iota(jnp.int32, SC_LANES)
    lane0 = lane == 0

    def do_chunk(k, j):
        wait(j)

        for r in range(RW):
            row = r0 + k * RW + r
            row_vec = jnp.full((SC_LANES,), 0, jnp.int32) + row
            rm = jnp.full((SC_LANES,), NEG_INF, jnp.float32)
            ra = jnp.zeros((SC_LANES,), jnp.int32)

            def seg(c, carry):
                srm, sra = carry
                off = c * SC_LANES
                v = buf_v[j, r, pl.ds(off, SC_LANES)]
                cm = cmax_v[pl.ds(off, SC_LANES)]
                ca = carg_v[pl.ds(off, SC_LANES)]
                m = v > cm
                cmax_v[pl.ds(off, SC_LANES)] = jnp.where(m, v, cm)
                carg_v[pl.ds(off, SC_LANES)] = jnp.where(m, row_vec, ca)
                mr = v > srm
                srm = jnp.where(mr, v, srm)
                sra = jnp.where(mr, off + lane, sra)
                return srm, sra

            rm, ra = lax.fori_loop(0, NSEG, seg, (rm, ra))

            rmm = lax.reduce_max(rm, axes=(0,))
            ram = lax.reduce_min(jnp.where(rm == rmm, ra, BIG_I32), axes=(0,))
            rl = k * RW + r
            idxv = jnp.zeros((SC_LANES,), jnp.int32) + rl
            plsc.store_scatter(rpm_v, [idxv], jnp.zeros((SC_LANES,), jnp.float32) + rmm, mask=lane0)
            plsc.store_scatter(rpa_v, [idxv], jnp.zeros((SC_LANES,), jnp.int32) + ram, mask=lane0)

        rl0 = k * RW
        rowbase_vec = jnp.zeros((SC_LANES,), jnp.int32) + (r0 + rl0) + lane
        v48 = plsc.load_gather(
            buf_v, [jnp.zeros((SC_LANES,), jnp.int32) + j, lane,
                    jnp.zeros((SC_LANES,), jnp.int32) + NO])
        pm16 = rpm_v[pl.ds(rl0, SC_LANES)]
        pa16 = rpa_v[pl.ds(rl0, SC_LANES)]
        m48 = v48 > pm16
        rpm_v[pl.ds(rl0, SC_LANES)] = jnp.where(m48, v48, pm16)
        rpa_v[pl.ds(rl0, SC_LANES)] = jnp.where(m48, jnp.zeros((SC_LANES,), jnp.int32) + NO, pa16)
        c48max = lax.reduce_max(v48, axes=(0,))
        c48arg = lax.reduce_min(
            jnp.where(v48 == c48max, rowbase_vec, BIG_I32), axes=(0,))
        cm = cmax_v[pl.ds(NO, SC_LANES)]
        ca = carg_v[pl.ds(NO, SC_LANES)]
        vm = jnp.where(lane0, jnp.zeros((SC_LANES,), jnp.float32) + c48max, NEG_INF)
        m = vm > cm
        cmax_v[pl.ds(NO, SC_LANES)] = jnp.where(m, vm, cm)
        carg_v[pl.ds(NO, SC_LANES)] = jnp.where(
            m, jnp.zeros((SC_LANES,), jnp.int32) + c48arg, ca)

        @pl.when(k + 2 < NCHUNK)
        def _():
            start(k + 2, j)

    def outer(kk, carry):
        do_chunk(2 * kk, 0)
        do_chunk(2 * kk + 1, 1)
        return carry

    lax.fori_loop(0, NCHUNK // 2, outer, 0)

    pltpu.sync_copy(cmax_v, colpm_hbm.at[wid])
    pltpu.sync_copy(carg_v, colpa_hbm.at[wid])
    pltpu.sync_copy(rpm_v, rowpm_hbm.at[p, pl.ds(q * RPW, RPW)])
    pltpu.sync_copy(rpa_v, rowpa_hbm.at[p, pl.ds(q * RPW, RPW)])


def _merge_body(tc_rm_ref, tc_ra_ref, tc_cm_ref, tc_ca_ref,
                scpm_ref, scpa_ref, srm_ref, sra_ref,
                max0_ref, m0_ref, m1_ref):
    bm = tc_cm_ref[:, :, 0, :].reshape(B, N)
    ba = tc_ca_ref[:, :, 0, :].reshape(B, N)
    scpm = scpm_ref[...].reshape(B, WPR, PAD)
    scpa = scpa_ref[...].reshape(B, WPR, PAD)
    for qq in range(WPR):
        cv = scpm[:, qq, :N]
        ci = scpa[:, qq, :N]
        take = jnp.logical_or(cv > bm, jnp.logical_and(cv == bm, ci < ba))
        bm = jnp.where(take, cv, bm)
        ba = jnp.where(take, ci, ba)
    m1_ref[:, 0:N] = ba
    m1_ref[:, N:PAD] = jnp.zeros((B, PAD - N), jnp.int32)

    tcrm = tc_rm_ref[:, :, 0, :].reshape(B, N)
    tcra = tc_ra_ref[:, :, 0, :].reshape(B, N)
    max0_ref[:, 0:RTC] = tcrm[:, 0:RTC]
    max0_ref[:, RTC:NO] = srm_ref[...]
    max0_ref[:, NO:PAD] = jnp.zeros((B, PAD - NO), jnp.float32)
    m0_ref[:, 0:RTC] = tcra[:, 0:RTC]
    m0_ref[:, RTC:NO] = sra_ref[...]
    m0_ref[:, NO:PAD] = jnp.zeros((B, PAD - NO), jnp.int32)


_merge = pl.pallas_call(
    _merge_body,
    out_shape=[
        jax.ShapeDtypeStruct((B, PAD), jnp.float32),
        jax.ShapeDtypeStruct((B, PAD), jnp.int32),
        jax.ShapeDtypeStruct((B, PAD), jnp.int32),
    ],
)


@functools.partial(
    pl.kernel,
    out_type=jax.ShapeDtypeStruct((B, NO), jnp.int32),
    mesh=plsc.VectorSubcoreMesh(core_axis_name="c", subcore_axis_name="s"),
    compiler_params=pltpu.CompilerParams(needs_layout_passes=False),
    scratch_types=[
        pltpu.VMEM((PAD,), jnp.int32),
        pltpu.VMEM((CPW,), jnp.int32),
        pltpu.VMEM((CPW,), jnp.float32),
        pltpu.VMEM((CPW,), jnp.int32),
    ],
)
def _stage2(max0_hbm, m0_hbm, m1_hbm, out_hbm, m1row_v, m0_v, mx_v, out_v):
    wid = lax.axis_index("s") * SC_CORES + lax.axis_index("c")
    p = wid // WPR
    base = (wid % WPR) * CPW
    pltpu.sync_copy(m1_hbm.at[p], m1row_v)
    pltpu.sync_copy(m0_hbm.at[p, pl.ds(base, CPW)], m0_v)
    pltpu.sync_copy(max0_hbm.at[p, pl.ds(base, CPW)], mx_v)
    for k in range(CPW // SC_LANES):
        off = k * SC_LANES
        idx = m0_v[pl.ds(off, SC_LANES)]
        g = plsc.load_gather(m1row_v, [idx])
        r = base + off + lax.iota(jnp.int32, SC_LANES)
        mutual = g == r
        ok = jnp.logical_and(mutual, mx_v[pl.ds(off, SC_LANES)] > MATCH_THRESHOLD_F32)
        out_v[pl.ds(off, SC_LANES)] = jnp.where(ok, idx, np.int32(-1))
    pltpu.sync_copy(out_v, out_hbm.at[p, pl.ds(base, CPW)])


@jax.jit
def kernel(scores_list):
    tc_rm, tc_ra, tc_cm, tc_ca = _stage1a(scores_list)
    sc_pm, sc_pa, sc_rm, sc_ra = _stage1b(scores_list)
    max0p, m0p, m1p = _merge(tc_rm, tc_ra, tc_cm, tc_ca, sc_pm, sc_pa, sc_rm, sc_ra)
    out = _stage2(max0p, m0p, m1p).reshape(2, 4, NO)
    return (out[0], out[1])


# speedup vs baseline: 1.0427x; 1.0427x over previous
import functools

import jax
import jax.numpy as jnp
import numpy as np
from jax import lax
from jax.experimental import pallas as pl
from jax.experimental.pallas import tpu as pltpu
from jax.experimental.pallas import tpu_sc as plsc

N = 2049
B = 8
TR = 256
PAD = 2064
NO = 2048

RTC = 1792
NT_FULL = RTC // TR
LAST_BLK = 2048 // TR

MATCH_THRESHOLD_F32 = np.float32(0.2)
BIG_I32 = np.int32(2**30)
NEG_INF = np.float32(-np.inf)

SC_CORES = 2
SC_SUBCORES = 16
SC_LANES = 16
NW = SC_CORES * SC_SUBCORES
WPR = NW // B
SROWS = NO - RTC
RPW = SROWS // WPR
RW = 16
NCHUNK = RPW // RW
NSEG = N // SC_LANES
TAIL = N - SC_LANES
CPW = NO // WPR


def _stage1a_body(x_ref, rowmax_ref, rowarg_ref, colmax_ref, colarg_ref):
    t = pl.program_id(2)
    x = x_ref[0, 0]

    col_ids = lax.broadcasted_iota(jnp.int32, (TR, N), 1)
    rmax = jnp.max(x, axis=1, keepdims=True)
    rarg = jnp.min(jnp.where(x == rmax, col_ids, BIG_I32), axis=1, keepdims=True)
    rowmax_ref[0, 0] = rmax.T
    rowarg_ref[0, 0] = rarg.T

    @pl.when(t == 0)
    def _():
        row_ids = lax.broadcasted_iota(jnp.int32, (TR, N), 0)
        cmax = jnp.max(x, axis=0, keepdims=True)
        carg = jnp.min(jnp.where(x == cmax, row_ids, BIG_I32), axis=0, keepdims=True)
        colmax_ref[0, 0] = cmax
        colarg_ref[0, 0] = carg

    @pl.when(jnp.logical_and(t > 0, t < NT_FULL))
    def _():
        row_ids = lax.broadcasted_iota(jnp.int32, (TR, N), 0) + t * TR
        cmax = jnp.max(x, axis=0, keepdims=True)
        carg = jnp.min(jnp.where(x == cmax, row_ids, BIG_I32), axis=0, keepdims=True)
        prev_max = colmax_ref[0, 0]
        prev_arg = colarg_ref[0, 0]
        upd = cmax > prev_max
        colmax_ref[0, 0] = jnp.where(upd, cmax, prev_max)
        colarg_ref[0, 0] = jnp.where(upd, carg, prev_arg)

    @pl.when(t == NT_FULL)
    def _():
        last = x[0:1, :]
        prev_max = colmax_ref[0, 0]
        prev_arg = colarg_ref[0, 0]
        upd = last > prev_max
        colmax_ref[0, 0] = jnp.where(upd, last, prev_max)
        colarg_ref[0, 0] = jnp.where(upd, jnp.full_like(prev_arg, N - 1), prev_arg)


def _row_blk(t):
    return jnp.where(t < NT_FULL, t, LAST_BLK)


_stage1a = pl.pallas_call(
    _stage1a_body,
    grid=(2, 4, NT_FULL + 1),
    in_specs=[pl.BlockSpec((1, 1, TR, N), lambda a, b, t: (a, b, _row_blk(t), 0))],
    out_specs=[
        pl.BlockSpec((1, 1, 1, TR), lambda a, b, t: (a, b, 0, _row_blk(t))),
        pl.BlockSpec((1, 1, 1, TR), lambda a, b, t: (a, b, 0, _row_blk(t))),
        pl.BlockSpec((1, 1, 1, N), lambda a, b, t: (a, b, 0, 0)),
        pl.BlockSpec((1, 1, 1, N), lambda a, b, t: (a, b, 0, 0)),
    ],
    out_shape=[
        jax.ShapeDtypeStruct((2, 4, 1, N), jnp.float32),
        jax.ShapeDtypeStruct((2, 4, 1, N), jnp.int32),
        jax.ShapeDtypeStruct((2, 4, 1, N), jnp.float32),
        jax.ShapeDtypeStruct((2, 4, 1, N), jnp.int32),
    ],
    compiler_params=pltpu.CompilerParams(
        dimension_semantics=("parallel", "parallel", "arbitrary"),
    ),
)


@functools.partial(
    pl.kernel,
    out_type=[
        jax.ShapeDtypeStruct((NW, PAD), jnp.float32),
        jax.ShapeDtypeStruct((NW, PAD), jnp.int32),
        jax.ShapeDtypeStruct((B, SROWS), jnp.float32),
        jax.ShapeDtypeStruct((B, SROWS), jnp.int32),
    ],
    mesh=plsc.VectorSubcoreMesh(core_axis_name="c", subcore_axis_name="s"),
    compiler_params=pltpu.CompilerParams(needs_layout_passes=False),
    scratch_types=[
        pltpu.VMEM((2, RW, N), jnp.float32),
        pltpu.VMEM((PAD,), jnp.float32),
        pltpu.VMEM((PAD,), jnp.int32),
        pltpu.VMEM((RPW,), jnp.float32),
        pltpu.VMEM((RPW,), jnp.int32),
        pltpu.SemaphoreType.DMA,
        pltpu.SemaphoreType.DMA,
    ],
)
def _stage1b(x_hbm, colpm_hbm, colpa_hbm, rowpm_hbm, rowpa_hbm,
             buf_v, cmax_v, carg_v, rpm_v, rpa_v, sem0, sem1):
    wid = lax.axis_index("s") * SC_CORES + lax.axis_index("c")
    p = wid // WPR
    q = wid % WPR
    a = p // 4
    b = p % 4
    r0 = RTC + q * RPW

    for c in range(PAD // SC_LANES):
        cmax_v[pl.ds(c * SC_LANES, SC_LANES)] = jnp.full((SC_LANES,), NEG_INF, jnp.float32)
        carg_v[pl.ds(c * SC_LANES, SC_LANES)] = jnp.zeros((SC_LANES,), jnp.int32)

    sems = (sem0, sem1)

    def start(k, j):
        pltpu.make_async_copy(
            x_hbm.at[a, b, pl.ds(r0 + k * RW, RW), :], buf_v.at[j], sems[j]).start()

    def wait(j):
        pltpu.make_async_copy(
            x_hbm.at[a, b, pl.ds(r0, RW), :], buf_v.at[j], sems[j]).wait()

    start(0, 0)
    start(np.int32(1), 1)

    lane = lax.iota(jnp.int32, SC_LANES)
    lane0 = lane == 0

    def do_chunk(k, j):
        wait(j)

        for r in range(RW):
            row = r0 + k * RW + r
            row_vec = jnp.full((SC_LANES,), 0, jnp.int32) + row
            rm = jnp.full((SC_LANES,), NEG_INF, jnp.float32)
            ra = jnp.zeros((SC_LANES,), jnp.int32)

            def seg(c, carry):
                srm, sra = carry
                off = c * SC_LANES
                v = buf_v[j, r, pl.ds(off, SC_LANES)]
                cm = cmax_v[pl.ds(off, SC_LANES)]
                ca = carg_v[pl.ds(off, SC_LANES)]
                m = v > cm
                cmax_v[pl.ds(off, SC_LANES)] = jnp.where(m, v, cm)
                carg_v[pl.ds(off, SC_LANES)] = jnp.where(m, row_vec, ca)
                mr = v > srm
                srm = jnp.where(mr, v, srm)
                sra = jnp.where(mr, off + lane, sra)
                return srm, sra

            rm, ra = lax.fori_loop(0, NSEG, seg, (rm, ra))

            rmm = lax.reduce_max(rm, axes=(0,))
            ram = lax.reduce_min(jnp.where(rm == rmm, ra, BIG_I32), axes=(0,))
            rl = k * RW + r
            idxv = jnp.zeros((SC_LANES,), jnp.int32) + rl
            plsc.store_scatter(rpm_v, [idxv], jnp.zeros((SC_LANES,), jnp.float32) + rmm, mask=lane0)
            plsc.store_scatter(rpa_v, [idxv], jnp.zeros((SC_LANES,), jnp.int32) + ram, mask=lane0)

        rl0 = k * RW
        rowbase_vec = jnp.zeros((SC_LANES,), jnp.int32) + (r0 + rl0) + lane
        v48 = plsc.load_gather(
            buf_v, [jnp.zeros((SC_LANES,), jnp.int32) + j, lane,
                    jnp.zeros((SC_LANES,), jnp.int32) + NO])
        pm16 = rpm_v[pl.ds(rl0, SC_LANES)]
        pa16 = rpa_v[pl.ds(rl0, SC_LANES)]
        m48 = v48 > pm16
        rpm_v[pl.ds(rl0, SC_LANES)] = jnp.where(m48, v48, pm16)
        rpa_v[pl.ds(rl0, SC_LANES)] = jnp.where(m48, jnp.zeros((SC_LANES,), jnp.int32) + NO, pa16)
        c48max = lax.reduce_max(v48, axes=(0,))
        c48arg = lax.reduce_min(
            jnp.where(v48 == c48max, rowbase_vec, BIG_I32), axes=(0,))
        cm = cmax_v[pl.ds(NO, SC_LANES)]
        ca = carg_v[pl.ds(NO, SC_LANES)]
        vm = jnp.where(lane0, jnp.zeros((SC_LANES,), jnp.float32) + c48max, NEG_INF)
        m = vm > cm
        cmax_v[pl.ds(NO, SC_LANES)] = jnp.where(m, vm, cm)
        carg_v[pl.ds(NO, SC_LANES)] = jnp.where(
            m, jnp.zeros((SC_LANES,), jnp.int32) + c48arg, ca)

        @pl.when(k + 2 < NCHUNK)
        def _():
            start(k + 2, j)

    def outer(kk, carry):
        do_chunk(2 * kk, 0)
        do_chunk(2 * kk + 1, 1)
        return carry

    lax.fori_loop(0, NCHUNK // 2, outer, 0)

    pltpu.sync_copy(cmax_v, colpm_hbm.at[wid])
    pltpu.sync_copy(carg_v, colpa_hbm.at[wid])
    pltpu.sync_copy(rpm_v, rowpm_hbm.at[p, pl.ds(q * RPW, RPW)])
    pltpu.sync_copy(rpa_v, rowpa_hbm.at[p, pl.ds(q * RPW, RPW)])


def _merge_body(tc_rm_ref, tc_ra_ref, tc_cm_ref, tc_ca_ref,
                scpm_ref, scpa_ref, srm_ref, sra_ref,
                max0_ref, m0_ref, m1_ref):
    bm = tc_cm_ref[:, :, 0, :].reshape(B, N)
    ba = tc_ca_ref[:, :, 0, :].reshape(B, N)
    scpm = scpm_ref[...].reshape(B, WPR, PAD)
    scpa = scpa_ref[...].reshape(B, WPR, PAD)
    for qq in range(WPR):
        cv = scpm[:, qq, :N]
        ci = scpa[:, qq, :N]
        take = jnp.logical_or(cv > bm, jnp.logical_and(cv == bm, ci < ba))
        bm = jnp.where(take, cv, bm)
        ba = jnp.where(take, ci, ba)
    m1_ref[:, 0:N] = ba
    m1_ref[:, N:PAD] = jnp.zeros((B, PAD - N), jnp.int32)

    tcrm = tc_rm_ref[:, :, 0, :].reshape(B, N)
    tcra = tc_ra_ref[:, :, 0, :].reshape(B, N)
    max0_ref[:, 0:RTC] = tcrm[:, 0:RTC]
    max0_ref[:, RTC:NO] = srm_ref[...]
    max0_ref[:, NO:PAD] = jnp.zeros((B, PAD - NO), jnp.float32)
    m0_ref[:, 0:RTC] = tcra[:, 0:RTC]
    m0_ref[:, RTC:NO] = sra_ref[...]
    m0_ref[:, NO:PAD] = jnp.zeros((B, PAD - NO), jnp.int32)


_merge = pl.pallas_call(
    _merge_body,
    out_shape=[
        jax.ShapeDtypeStruct((B, PAD), jnp.float32),
        jax.ShapeDtypeStruct((B, PAD), jnp.int32),
        jax.ShapeDtypeStruct((B, PAD), jnp.int32),
    ],
)


@functools.partial(
    pl.kernel,
    out_type=jax.ShapeDtypeStruct((B, NO), jnp.int32),
    mesh=plsc.VectorSubcoreMesh(core_axis_name="c", subcore_axis_name="s"),
    compiler_params=pltpu.CompilerParams(needs_layout_passes=False),
    scratch_types=[
        pltpu.VMEM((PAD,), jnp.int32),
        pltpu.VMEM((CPW,), jnp.int32),
        pltpu.VMEM((CPW,), jnp.float32),
        pltpu.VMEM((CPW,), jnp.int32),
    ],
)
def _stage2(max0_hbm, m0_hbm, m1_hbm, out_hbm, m1row_v, m0_v, mx_v, out_v):
    wid = lax.axis_index("s") * SC_CORES + lax.axis_index("c")
    p = wid // WPR
    base = (wid % WPR) * CPW
    pltpu.sync_copy(m1_hbm.at[p], m1row_v)
    pltpu.sync_copy(m0_hbm.at[p, pl.ds(base, CPW)], m0_v)
    pltpu.sync_copy(max0_hbm.at[p, pl.ds(base, CPW)], mx_v)
    for k in range(CPW // SC_LANES):
        off = k * SC_LANES
        idx = m0_v[pl.ds(off, SC_LANES)]
        g = plsc.load_gather(m1row_v, [idx])
        r = base + off + lax.iota(jnp.int32, SC_LANES)
        mutual = g == r
        ok = jnp.logical_and(mutual, mx_v[pl.ds(off, SC_LANES)] > MATCH_THRESHOLD_F32)
        out_v[pl.ds(off, SC_LANES)] = jnp.where(ok, idx, np.int32(-1))
    pltpu.sync_copy(out_v, out_hbm.at[p, pl.ds(base, CPW)])


@jax.jit
def kernel(scores_list):
    tc_rm, tc_ra, tc_cm, tc_ca = _stage1a(scores_list)
    sc_pm, sc_pa, sc_rm, sc_ra = _stage1b(scores_list)
    max0p, m0p, m1p = _merge(tc_rm, tc_ra, tc_cm, tc_ca, sc_pm, sc_pa, sc_rm, sc_ra)
    out = _stage2(max0p, m0p, m1p).reshape(2, 4, NO)
    return (out[0], out[1])


# speedup vs baseline: 1.1577x; 1.1103x over previous
import functools

import jax
import jax.numpy as jnp
import numpy as np
from jax import lax
from jax.experimental import pallas as pl
from jax.experimental.pallas import tpu as pltpu
from jax.experimental.pallas import tpu_sc as plsc

N = 2049
B = 8
TR = 512
PAD = 2064
NO = 2048

RTC = 1536
NT_FULL = RTC // TR
LAST_BLK = 2048 // TR

MATCH_THRESHOLD_F32 = np.float32(0.2)
BIG_I32 = np.int32(2**30)
NEG_INF = np.float32(-np.inf)

SC_CORES = 2
SC_SUBCORES = 16
SC_LANES = 16
NW = SC_CORES * SC_SUBCORES
WPR = NW // B
SROWS = NO - RTC
RPW = SROWS // WPR
RW = 16
NCHUNK = RPW // RW
NSEG = N // SC_LANES
TAIL = N - SC_LANES
CPW = NO // WPR


def _stage1a_body(x_ref, rowmax_ref, rowarg_ref, colmax_ref, colarg_ref):
    t = pl.program_id(2)
    x = x_ref[0, 0]

    col_ids = lax.broadcasted_iota(jnp.int32, (TR, N), 1)
    rmax = jnp.max(x, axis=1, keepdims=True)
    rarg = jnp.min(jnp.where(x == rmax, col_ids, BIG_I32), axis=1, keepdims=True)
    rowmax_ref[0, 0] = rmax.T
    rowarg_ref[0, 0] = rarg.T

    @pl.when(t == 0)
    def _():
        row_ids = lax.broadcasted_iota(jnp.int32, (TR, N), 0)
        cmax = jnp.max(x, axis=0, keepdims=True)
        carg = jnp.min(jnp.where(x == cmax, row_ids, BIG_I32), axis=0, keepdims=True)
        colmax_ref[0, 0] = cmax
        colarg_ref[0, 0] = carg

    @pl.when(jnp.logical_and(t > 0, t < NT_FULL))
    def _():
        row_ids = lax.broadcasted_iota(jnp.int32, (TR, N), 0) + t * TR
        cmax = jnp.max(x, axis=0, keepdims=True)
        carg = jnp.min(jnp.where(x == cmax, row_ids, BIG_I32), axis=0, keepdims=True)
        prev_max = colmax_ref[0, 0]
        prev_arg = colarg_ref[0, 0]
        upd = cmax > prev_max
        colmax_ref[0, 0] = jnp.where(upd, cmax, prev_max)
        colarg_ref[0, 0] = jnp.where(upd, carg, prev_arg)

    @pl.when(t == NT_FULL)
    def _():
        last = x[0:1, :]
        prev_max = colmax_ref[0, 0]
        prev_arg = colarg_ref[0, 0]
        upd = last > prev_max
        colmax_ref[0, 0] = jnp.where(upd, last, prev_max)
        colarg_ref[0, 0] = jnp.where(upd, jnp.full_like(prev_arg, N - 1), prev_arg)


def _row_blk(t):
    return jnp.where(t < NT_FULL, t, LAST_BLK)


_stage1a = pl.pallas_call(
    _stage1a_body,
    grid=(2, 4, NT_FULL + 1),
    in_specs=[pl.BlockSpec((1, 1, TR, N), lambda a, b, t: (a, b, _row_blk(t), 0))],
    out_specs=[
        pl.BlockSpec((1, 1, 1, TR), lambda a, b, t: (a, b, 0, _row_blk(t))),
        pl.BlockSpec((1, 1, 1, TR), lambda a, b, t: (a, b, 0, _row_blk(t))),
        pl.BlockSpec((1, 1, 1, N), lambda a, b, t: (a, b, 0, 0)),
        pl.BlockSpec((1, 1, 1, N), lambda a, b, t: (a, b, 0, 0)),
    ],
    out_shape=[
        jax.ShapeDtypeStruct((2, 4, 1, N), jnp.float32),
        jax.ShapeDtypeStruct((2, 4, 1, N), jnp.int32),
        jax.ShapeDtypeStruct((2, 4, 1, N), jnp.float32),
        jax.ShapeDtypeStruct((2, 4, 1, N), jnp.int32),
    ],
    compiler_params=pltpu.CompilerParams(
        dimension_semantics=("parallel", "parallel", "arbitrary"),
    ),
)


@functools.partial(
    pl.kernel,
    out_type=[
        jax.ShapeDtypeStruct((NW, PAD), jnp.float32),
        jax.ShapeDtypeStruct((NW, PAD), jnp.int32),
        jax.ShapeDtypeStruct((B, SROWS), jnp.float32),
        jax.ShapeDtypeStruct((B, SROWS), jnp.int32),
    ],
    mesh=plsc.VectorSubcoreMesh(core_axis_name="c", subcore_axis_name="s"),
    compiler_params=pltpu.CompilerParams(needs_layout_passes=False),
    scratch_types=[
        pltpu.VMEM((2, RW, N), jnp.float32),
        pltpu.VMEM((PAD,), jnp.float32),
        pltpu.VMEM((PAD,), jnp.int32),
        pltpu.VMEM((RPW,), jnp.float32),
        pltpu.VMEM((RPW,), jnp.int32),
        pltpu.SemaphoreType.DMA,
        pltpu.SemaphoreType.DMA,
    ],
)
def _stage1b(x_hbm, colpm_hbm, colpa_hbm, rowpm_hbm, rowpa_hbm,
             buf_v, cmax_v, carg_v, rpm_v, rpa_v, sem0, sem1):
    wid = lax.axis_index("s") * SC_CORES + lax.axis_index("c")
    p = wid // WPR
    q = wid % WPR
    a = p // 4
    b = p % 4
    r0 = RTC + q * RPW

    for c in range(PAD // SC_LANES):
        cmax_v[pl.ds(c * SC_LANES, SC_LANES)] = jnp.full((SC_LANES,), NEG_INF, jnp.float32)
        carg_v[pl.ds(c * SC_LANES, SC_LANES)] = jnp.zeros((SC_LANES,), jnp.int32)

    sems = (sem0, sem1)

    def start(k, j):
        pltpu.make_async_copy(
            x_hbm.at[a, b, pl.ds(r0 + k * RW, RW), :], buf_v.at[j], sems[j]).start()

    def wait(j):
        pltpu.make_async_copy(
            x_hbm.at[a, b, pl.ds(r0, RW), :], buf_v.at[j], sems[j]).wait()

    start(0, 0)
    start(np.int32(1), 1)

    lane = lax.iota(jnp.int32, SC_LANES)
    lane0 = lane == 0

    def do_chunk(k, j):
        wait(j)

        for r in range(0, RW, 4):
            rows = [r0 + k * RW + r + i for i in range(4)]
            row_vecs = [jnp.full((SC_LANES,), 0, jnp.int32) + rr for rr in rows]
            init = (jnp.full((SC_LANES,), NEG_INF, jnp.float32),
                    jnp.zeros((SC_LANES,), jnp.int32))

            def seg(c, carry):
                (rm0, ra0), (rm1, ra1), (rm2, ra2), (rm3, ra3) = carry
                off = c * SC_LANES
                cid = off + lane
                cm = cmax_v[pl.ds(off, SC_LANES)]
                ca = carg_v[pl.ds(off, SC_LANES)]
                outs = []
                for i, (srm, sra) in enumerate(((rm0, ra0), (rm1, ra1),
                                                (rm2, ra2), (rm3, ra3))):
                    v = buf_v[j, r + i, pl.ds(off, SC_LANES)]
                    m = v > cm
                    cm = jnp.where(m, v, cm)
                    ca = jnp.where(m, row_vecs[i], ca)
                    mr = v > srm
                    outs.append((jnp.where(mr, v, srm), jnp.where(mr, cid, sra)))
                cmax_v[pl.ds(off, SC_LANES)] = cm
                carg_v[pl.ds(off, SC_LANES)] = ca
                return tuple(outs)

            res = lax.fori_loop(0, NSEG, seg, (init, init, init, init))

            for i in range(4):
                rm, ra = res[i]
                rmm = lax.reduce_max(rm, axes=(0,))
                ram = lax.reduce_min(jnp.where(rm == rmm, ra, BIG_I32), axes=(0,))
                rl = k * RW + r + i
                idxv = jnp.zeros((SC_LANES,), jnp.int32) + rl
                plsc.store_scatter(rpm_v, [idxv], jnp.zeros((SC_LANES,), jnp.float32) + rmm, mask=lane0)
                plsc.store_scatter(rpa_v, [idxv], jnp.zeros((SC_LANES,), jnp.int32) + ram, mask=lane0)

        rl0 = k * RW
        rowbase_vec = jnp.zeros((SC_LANES,), jnp.int32) + (r0 + rl0) + lane
        v48 = plsc.load_gather(
            buf_v, [jnp.zeros((SC_LANES,), jnp.int32) + j, lane,
                    jnp.zeros((SC_LANES,), jnp.int32) + NO])
        pm16 = rpm_v[pl.ds(rl0, SC_LANES)]
        pa16 = rpa_v[pl.ds(rl0, SC_LANES)]
        m48 = v48 > pm16
        rpm_v[pl.ds(rl0, SC_LANES)] = jnp.where(m48, v48, pm16)
        rpa_v[pl.ds(rl0, SC_LANES)] = jnp.where(m48, jnp.zeros((SC_LANES,), jnp.int32) + NO, pa16)
        c48max = lax.reduce_max(v48, axes=(0,))
        c48arg = lax.reduce_min(
            jnp.where(v48 == c48max, rowbase_vec, BIG_I32), axes=(0,))
        cm = cmax_v[pl.ds(NO, SC_LANES)]
        ca = carg_v[pl.ds(NO, SC_LANES)]
        vm = jnp.where(lane0, jnp.zeros((SC_LANES,), jnp.float32) + c48max, NEG_INF)
        m = vm > cm
        cmax_v[pl.ds(NO, SC_LANES)] = jnp.where(m, vm, cm)
        carg_v[pl.ds(NO, SC_LANES)] = jnp.where(
            m, jnp.zeros((SC_LANES,), jnp.int32) + c48arg, ca)

        @pl.when(k + 2 < NCHUNK)
        def _():
            start(k + 2, j)

    def outer(kk, carry):
        do_chunk(2 * kk, 0)
        do_chunk(2 * kk + 1, 1)
        return carry

    lax.fori_loop(0, NCHUNK // 2, outer, 0)

    pltpu.sync_copy(cmax_v, colpm_hbm.at[wid])
    pltpu.sync_copy(carg_v, colpa_hbm.at[wid])
    pltpu.sync_copy(rpm_v, rowpm_hbm.at[p, pl.ds(q * RPW, RPW)])
    pltpu.sync_copy(rpa_v, rowpa_hbm.at[p, pl.ds(q * RPW, RPW)])


def _merge_body(tc_rm_ref, tc_ra_ref, tc_cm_ref, tc_ca_ref,
                scpm_ref, scpa_ref, srm_ref, sra_ref,
                max0_ref, m0_ref, m1_ref):
    bm = tc_cm_ref[:, :, 0, :].reshape(B, N)
    ba = tc_ca_ref[:, :, 0, :].reshape(B, N)
    scpm = scpm_ref[...].reshape(B, WPR, PAD)
    scpa = scpa_ref[...].reshape(B, WPR, PAD)
    for qq in range(WPR):
        cv = scpm[:, qq, :N]
        ci = scpa[:, qq, :N]
        take = jnp.logical_or(cv > bm, jnp.logical_and(cv == bm, ci < ba))
        bm = jnp.where(take, cv, bm)
        ba = jnp.where(take, ci, ba)
    m1_ref[:, 0:N] = ba
    m1_ref[:, N:PAD] = jnp.zeros((B, PAD - N), jnp.int32)

    tcrm = tc_rm_ref[:, :, 0, :].reshape(B, N)
    tcra = tc_ra_ref[:, :, 0, :].reshape(B, N)
    max0_ref[:, 0:RTC] = tcrm[:, 0:RTC]
    max0_ref[:, RTC:NO] = srm_ref[...]
    max0_ref[:, NO:PAD] = jnp.zeros((B, PAD - NO), jnp.float32)
    m0_ref[:, 0:RTC] = tcra[:, 0:RTC]
    m0_ref[:, RTC:NO] = sra_ref[...]
    m0_ref[:, NO:PAD] = jnp.zeros((B, PAD - NO), jnp.int32)


_merge = pl.pallas_call(
    _merge_body,
    out_shape=[
        jax.ShapeDtypeStruct((B, PAD), jnp.float32),
        jax.ShapeDtypeStruct((B, PAD), jnp.int32),
        jax.ShapeDtypeStruct((B, PAD), jnp.int32),
    ],
)


@functools.partial(
    pl.kernel,
    out_type=jax.ShapeDtypeStruct((B, NO), jnp.int32),
    mesh=plsc.VectorSubcoreMesh(core_axis_name="c", subcore_axis_name="s"),
    compiler_params=pltpu.CompilerParams(needs_layout_passes=False),
    scratch_types=[
        pltpu.VMEM((PAD,), jnp.int32),
        pltpu.VMEM((CPW,), jnp.int32),
        pltpu.VMEM((CPW,), jnp.float32),
        pltpu.VMEM((CPW,), jnp.int32),
    ],
)
def _stage2(max0_hbm, m0_hbm, m1_hbm, out_hbm, m1row_v, m0_v, mx_v, out_v):
    wid = lax.axis_index("s") * SC_CORES + lax.axis_index("c")
    p = wid // WPR
    base = (wid % WPR) * CPW
    pltpu.sync_copy(m1_hbm.at[p], m1row_v)
    pltpu.sync_copy(m0_hbm.at[p, pl.ds(base, CPW)], m0_v)
    pltpu.sync_copy(max0_hbm.at[p, pl.ds(base, CPW)], mx_v)
    for k in range(CPW // SC_LANES):
        off = k * SC_LANES
        idx = m0_v[pl.ds(off, SC_LANES)]
        g = plsc.load_gather(m1row_v, [idx])
        r = base + off + lax.iota(jnp.int32, SC_LANES)
        mutual = g == r
        ok = jnp.logical_and(mutual, mx_v[pl.ds(off, SC_LANES)] > MATCH_THRESHOLD_F32)
        out_v[pl.ds(off, SC_LANES)] = jnp.where(ok, idx, np.int32(-1))
    pltpu.sync_copy(out_v, out_hbm.at[p, pl.ds(base, CPW)])


@jax.jit
def kernel(scores_list):
    tc_rm, tc_ra, tc_cm, tc_ca = _stage1a(scores_list)
    sc_pm, sc_pa, sc_rm, sc_ra = _stage1b(scores_list)
    max0p, m0p, m1p = _merge(tc_rm, tc_ra, tc_cm, tc_ca, sc_pm, sc_pa, sc_rm, sc_ra)
    out = _stage2(max0p, m0p, m1p).reshape(2, 4, NO)
    return (out[0], out[1])
